# baseline (speedup 1.0000x reference)
import jax
import jax.numpy as jnp
from jax.experimental import pallas as pl
from jax.experimental.pallas import tpu as pltpu

G, V, D, P = 8, 64, 64, 32


def _main_body(x_ref, wpre_ref, bpre_ref, wwp_ref, bwp_ref, cb_ref,
               wpost_ref, bpost_ref, out_ref, m_ref):
    @pl.when(pl.program_id(0) == 0)
    def _fuse_table():
        for g in range(G):
            m_ref[g * V:(g + 1) * V, :] = jnp.dot(
                cb_ref[g * V:(g + 1) * V, :],
                wpost_ref[g * V:(g + 1) * V, :],
                preferred_element_type=jnp.float32).astype(jnp.bfloat16)

    h = jnp.dot(x_ref[...], wpre_ref[...]) + bpre_ref[...]
    logits = jnp.dot(h, wwp_ref[...]) + bwp_ref[...]
    ohs = []
    for g in range(G):
        lg = logits[:, g * V:(g + 1) * V]
        mx = jnp.max(lg, axis=1, keepdims=True)
        ohs.append(jnp.where(lg >= mx, 1.0, 0.0))
    oh = jnp.concatenate(ohs, axis=1).astype(jnp.bfloat16)
    out_ref[...] = (jnp.dot(oh, m_ref[...],
                            preferred_element_type=jnp.float32)
                    + bpost_ref[...])


def kernel(x, W_pre, b_pre, W_wp, b_wp, codebook, W_post, b_post):
    B, T, IN = x.shape
    BT = B * T
    OUT = W_post.shape[1]
    GV = G * V

    R = 3072
    x2 = x.reshape(BT, IN)
    out = pl.pallas_call(
        _main_body,
        grid=(BT // R,),
        in_specs=[
            pl.BlockSpec((R, IN), lambda i: (i, 0)),
            pl.BlockSpec((IN, P), lambda i: (0, 0)),
            pl.BlockSpec((1, P), lambda i: (0, 0)),
            pl.BlockSpec((P, GV), lambda i: (0, 0)),
            pl.BlockSpec((1, GV), lambda i: (0, 0)),
            pl.BlockSpec((GV, D), lambda i: (0, 0)),
            pl.BlockSpec((GV, OUT), lambda i: (0, 0)),
            pl.BlockSpec((1, OUT), lambda i: (0, 0)),
        ],
        out_specs=pl.BlockSpec((R, OUT), lambda i: (i, 0)),
        out_shape=jax.ShapeDtypeStruct((BT, OUT), jnp.float32),
        scratch_shapes=[pltpu.VMEM((GV, OUT), jnp.bfloat16)],
    )(x2, W_pre, b_pre.reshape(1, P), W_wp, b_wp.reshape(1, GV), codebook,
      W_post, b_post.reshape(1, OUT))
    return out.reshape(B, T, OUT)


# speedup vs baseline: 1.0416x; 1.0416x over previous
import jax
import jax.numpy as jnp
from jax.experimental import pallas as pl
from jax.experimental.pallas import tpu as pltpu

G, V, D, P = 8, 64, 64, 32


def _main_body(x_ref, wpre_ref, bpre_ref, wwp_ref, bwp_ref, cb_ref,
               wpost_ref, bpost_ref, out_ref, m_ref):
    @pl.when(pl.program_id(0) == 0)
    def _fuse_table():
        for g in range(G):
            m_ref[g * V:(g + 1) * V, :] = jnp.dot(
                cb_ref[g * V:(g + 1) * V, :],
                wpost_ref[g * V:(g + 1) * V, :],
                preferred_element_type=jnp.float32).astype(jnp.bfloat16)

    h = jnp.dot(x_ref[...], wpre_ref[...]) + bpre_ref[...]
    logits = jnp.dot(h, wwp_ref[...]) + bwp_ref[...]
    ohs = []
    for g in range(G):
        lg = logits[:, g * V:(g + 1) * V]
        mx = jnp.max(lg, axis=1, keepdims=True)
        ohs.append(jnp.where(lg >= mx, 1.0, 0.0))
    oh = jnp.concatenate(ohs, axis=1).astype(jnp.bfloat16)
    out_ref[...] = (jnp.dot(oh, m_ref[...],
                            preferred_element_type=jnp.float32)
                    + bpost_ref[...])


def kernel(x, W_pre, b_pre, W_wp, b_wp, codebook, W_post, b_post):
    B, T, IN = x.shape
    BT = B * T
    OUT = W_post.shape[1]
    GV = G * V

    R = 1536
    x2 = x.reshape(BT, IN)
    out = pl.pallas_call(
        _main_body,
        grid=(BT // R,),
        in_specs=[
            pl.BlockSpec((R, IN), lambda i: (i, 0)),
            pl.BlockSpec((IN, P), lambda i: (0, 0)),
            pl.BlockSpec((1, P), lambda i: (0, 0)),
            pl.BlockSpec((P, GV), lambda i: (0, 0)),
            pl.BlockSpec((1, GV), lambda i: (0, 0)),
            pl.BlockSpec((GV, D), lambda i: (0, 0)),
            pl.BlockSpec((GV, OUT), lambda i: (0, 0)),
            pl.BlockSpec((1, OUT), lambda i: (0, 0)),
        ],
        out_specs=pl.BlockSpec((R, OUT), lambda i: (i, 0)),
        out_shape=jax.ShapeDtypeStruct((BT, OUT), jnp.float32),
        scratch_shapes=[pltpu.VMEM((GV, OUT), jnp.bfloat16)],
    )(x2, W_pre, b_pre.reshape(1, P), W_wp, b_wp.reshape(1, GV), codebook,
      W_post, b_post.reshape(1, OUT))
    return out.reshape(B, T, OUT)
